# Initial kernel scaffold; baseline (speedup 1.0000x reference)
import jax
import jax.numpy as jnp
from jax.experimental import pallas as pl


def kernel(hidden_states, target_ids, W_row, b_row, col_weight, col_bias):
    raise NotImplementedError("write your pallas kernel here")


# speedup vs baseline: 4.7099x; 4.7099x over previous
import jax
import jax.numpy as jnp
from jax.experimental import pallas as pl
from jax.experimental.pallas import tpu as pltpu

_TS = 128
_D = 768
_N = 2048
_TM = 128
_NT = _N // _TM
_NSEG = 144
_NB = 16


def _route_kernel(tid_ref, hs_ref, wrow_ref, brow_ref,
                  hss_ref, cols_ref, lrow_ref, meta_ref):
    tid = tid_ref[:]
    rows = tid // _TS
    cols = tid % _TS
    hs_b = hs_ref[:].astype(jnp.bfloat16)

    rlog = jax.lax.dot_general(
        hs_b, wrow_ref[:].astype(jnp.bfloat16),
        (((1,), (1,)), ((), ())), preferred_element_type=jnp.float32
    ) + brow_ref[:]
    lane_ts = jax.lax.broadcasted_iota(jnp.int32, (_N, _TS), 1)
    oh_f = (rows == lane_ts).astype(jnp.float32)
    mx = jnp.max(rlog, axis=1, keepdims=True)
    lse = mx[:, 0] + jnp.log(jnp.sum(jnp.exp(rlog - mx), axis=1))
    tgt = jnp.sum(oh_f * rlog, axis=1)
    lrow_ref[:] = jnp.reshape(jnp.sum(lse - tgt) * (1.0 / _N), (1, 1))

    offs = jnp.sum((rows < lane_ts).astype(jnp.float32), axis=0,
                   keepdims=True)
    gpos_col = jax.lax.broadcasted_iota(jnp.int32, (_N, 1), 0)
    gpos_f = gpos_col.astype(jnp.float32)
    rs = jnp.sum(((gpos_f - offs) >= 0.0).astype(jnp.float32),
                 axis=1, keepdims=True) - 1.0
    rs_prev = jnp.sum(((gpos_f - 1.0 - offs) >= 0.0).astype(jnp.float32),
                      axis=1, keepdims=True) - 1.0
    isnew_f = jnp.maximum((gpos_col % _TM == 0).astype(jnp.float32),
                          (rs != rs_prev).astype(jnp.float32))

    lane_u = jax.lax.broadcasted_iota(jnp.int32, (_N, _TS + 8), 1)
    oh_ext_f = (rows == lane_u).astype(jnp.float32)
    u = (oh_ext_f
         + (lane_u == _TS).astype(jnp.float32) * isnew_f).astype(jnp.bfloat16)
    io_i = jax.lax.broadcasted_iota(jnp.int32, (_TM, _TM), 0)
    io_j = jax.lax.broadcasted_iota(jnp.int32, (_TM, _TM), 1)
    Lt = (io_i >= io_j).astype(jnp.bfloat16)
    blocks = []
    carry = jnp.zeros((1, _TS + 8), jnp.float32)
    for b in range(_NB):
        ub = u[b * _TM:(b + 1) * _TM, :]
        cb = jax.lax.dot_general(Lt, ub, (((1,), (0,)), ((), ())),
                                 preferred_element_type=jnp.float32) + carry
        blocks.append(cb)
        carry = cb[_TM - 1:_TM, :]
    CU = jnp.concatenate(blocks, axis=0)

    rank = jnp.sum(oh_f * CU[:, :_TS], axis=1, keepdims=True) - 1.0
    segid = CU[:, _TS:_TS + 1] - 1.0
    offs_own = jnp.sum(oh_f * offs, axis=1, keepdims=True)
    pos = offs_own + rank

    io_pos = jax.lax.broadcasted_iota(jnp.int32, (_N, _N), 1)
    PT = (pos == io_pos.astype(jnp.float32)).astype(jnp.bfloat16)
    hss_ref[:] = jax.lax.dot_general(
        PT, hs_b, (((0,), (0,)), ((), ())),
        preferred_element_type=jnp.float32).astype(jnp.bfloat16)
    lane_m = jax.lax.broadcasted_iota(jnp.int32, (_N, _TS), 1)
    Xs = ((lane_m == 0).astype(jnp.float32)
          * cols.astype(jnp.float32)).astype(jnp.bfloat16)
    colss = jax.lax.dot_general(PT, Xs, (((0,), (0,)), ((), ())),
                                preferred_element_type=jnp.float32)
    cols_ref[:] = colss[:, 0:1].astype(jnp.int32)

    lane_sg = jax.lax.broadcasted_iota(jnp.int32, (_N, _NSEG), 1)
    Qc = (segid == lane_sg.astype(jnp.float32)).astype(jnp.bfloat16)
    Qt = Qc * isnew_f.astype(jnp.bfloat16)
    ihi = (gpos_col // 16).astype(jnp.float32)
    ilo = (gpos_col % 16).astype(jnp.float32)
    l0 = (lane_m == 0).astype(jnp.float32)
    l1 = (lane_m == 1).astype(jnp.float32)
    l2 = (lane_m == 2).astype(jnp.float32)
    l3 = (lane_m == 3).astype(jnp.float32)
    M1 = (l0 * ihi + l1 * ilo + l2 * rs + l3).astype(jnp.bfloat16)
    o1 = jax.lax.dot_general(Qt, M1, (((0,), (0,)), ((), ())),
                             preferred_element_type=jnp.float32)
    o2 = jax.lax.dot_general(Qc, M1, (((0,), (0,)), ((), ())),
                             preferred_element_type=jnp.float32)
    start = 16.0 * o1[:, 0:1] + o1[:, 1:2]
    expert = o1[:, 2:3]
    count = o2[:, 3:4]
    end = start + count
    tile = jnp.floor(start * (1.0 / _TM))
    pad_f = (count == 0.0).astype(jnp.float32)
    tile = pad_f * float(_NT - 1) + (1.0 - pad_f) * tile
    expert = (1.0 - pad_f) * expert
    lane_meta = jax.lax.broadcasted_iota(jnp.int32, (_NSEG, _TS), 1)
    s0 = (lane_meta == 0).astype(jnp.float32)
    s1 = (lane_meta == 1).astype(jnp.float32)
    s2 = (lane_meta == 2).astype(jnp.float32)
    s3 = (lane_meta == 3).astype(jnp.float32)
    meta = s0 * expert + s1 * tile + s2 * start + s3 * end
    meta_ref[:] = meta.astype(jnp.int32)


def _group_kernel(se_ref, st_ref, ss_ref, sn_ref,
                  hss_ref, cw_ref, cb_ref, colss_ref, out_ref, acc_ref):
    s = pl.program_id(0)
    start = ss_ref[s]
    end = sn_ref[s]
    tile = st_ref[s]
    gpos = tile * _TM + jax.lax.broadcasted_iota(jnp.int32, (_TM, 1), 0)
    m = jnp.logical_and(gpos >= start, gpos < end)
    contrib = jnp.dot(hss_ref[:], cw_ref[0].astype(jnp.bfloat16),
                      preferred_element_type=jnp.float32) + cb_ref[0]
    val = jnp.where(m, contrib, 0.0)
    prev_tile = st_ref[jnp.maximum(s - 1, 0)]
    first = jnp.logical_or(s == 0, tile != prev_tile)
    next_tile = st_ref[jnp.minimum(s + 1, _NSEG - 1)]
    last = jnp.logical_or(s == _NSEG - 1, tile != next_tile)

    @pl.when(first)
    def _init():
        acc_ref[:] = val

    @pl.when(jnp.logical_not(first))
    def _accum():
        acc_ref[:] += val

    @pl.when(s == 0)
    def _zero_out():
        out_ref[:] = jnp.zeros_like(out_ref)

    @pl.when(last)
    def _tile_ce():
        clog = acc_ref[:]
        iot = jax.lax.broadcasted_iota(jnp.int32, (_TM, _TS), 1)
        onehot_f = (colss_ref[:] == iot).astype(jnp.float32)
        mx = jnp.max(clog, axis=1, keepdims=True)
        lse = mx[:, 0] + jnp.log(jnp.sum(jnp.exp(clog - mx), axis=1))
        tgt = jnp.sum(onehot_f * clog, axis=1)
        out_ref[:] += jnp.reshape(jnp.sum(lse - tgt) * (1.0 / _N), (1, 1))


def kernel(hidden_states, target_ids, W_row, b_row, col_weight, col_bias):
    hs = hidden_states.reshape(_N, _D)
    tid = target_ids.reshape(_N, 1)
    brow = b_row.reshape(1, _TS)

    hss, colss, lrow, meta = pl.pallas_call(
        _route_kernel,
        in_specs=[
            pl.BlockSpec((_N, 1), lambda: (0, 0)),
            pl.BlockSpec((_N, _D), lambda: (0, 0)),
            pl.BlockSpec((_TS, _D), lambda: (0, 0)),
            pl.BlockSpec((1, _TS), lambda: (0, 0)),
        ],
        out_specs=[
            pl.BlockSpec((_N, _D), lambda: (0, 0)),
            pl.BlockSpec((_N, 1), lambda: (0, 0)),
            pl.BlockSpec((1, 1), lambda: (0, 0)),
            pl.BlockSpec((_NSEG, _TS), lambda: (0, 0)),
        ],
        out_shape=[
            jax.ShapeDtypeStruct((_N, _D), jnp.bfloat16),
            jax.ShapeDtypeStruct((_N, 1), jnp.int32),
            jax.ShapeDtypeStruct((1, 1), jnp.float32),
            jax.ShapeDtypeStruct((_NSEG, _TS), jnp.int32),
        ],
    )(tid, hs, W_row, brow)

    se = meta[:, 0]
    st = meta[:, 1]
    ss = meta[:, 2]
    sn = meta[:, 3]

    lcol = pl.pallas_call(
        _group_kernel,
        grid_spec=pltpu.PrefetchScalarGridSpec(
            num_scalar_prefetch=4,
            grid=(_NSEG,),
            in_specs=[
                pl.BlockSpec((_TM, _D), lambda s, se, st, ss, sn: (st[s], 0)),
                pl.BlockSpec((1, _D, _TS),
                             lambda s, se, st, ss, sn: (se[s], 0, 0)),
                pl.BlockSpec((1, 1, _TS),
                             lambda s, se, st, ss, sn: (se[s], 0, 0)),
                pl.BlockSpec((_TM, 1), lambda s, se, st, ss, sn: (st[s], 0)),
            ],
            out_specs=pl.BlockSpec((1, 1), lambda s, se, st, ss, sn: (0, 0)),
            scratch_shapes=[pltpu.VMEM((_TM, _TS), jnp.float32)],
        ),
        out_shape=jax.ShapeDtypeStruct((1, 1), jnp.float32),
    )(se, st, ss, sn, hss, col_weight, col_bias.reshape(_TS, 1, _TS), colss)

    return (lrow + lcol)[0, 0]


# speedup vs baseline: 8.3597x; 1.7749x over previous
import jax
import jax.numpy as jnp
from jax.experimental import pallas as pl
from jax.experimental.pallas import tpu as pltpu

_TS = 128
_D = 768
_N = 2048
_TM = 256
_NT = _N // _TM
_K = 4
_NQ = _TS // _K
_NSEG = 40
_NB = 16
_CB = _N // _NB
_SENT = 1000


def _route_kernel(tid_ref, hs_ref, wrow_ref, brow_ref,
                  hss_ref, rows_ref, cols_ref, lrow_ref, meta_ref):
    tid = tid_ref[:]
    rows = tid // _TS
    cols = tid % _TS
    hs_b = hs_ref[:].astype(jnp.bfloat16)

    rlog = jax.lax.dot_general(
        hs_b, wrow_ref[:].astype(jnp.bfloat16),
        (((1,), (1,)), ((), ())), preferred_element_type=jnp.float32
    ) + brow_ref[:]
    lane_ts = jax.lax.broadcasted_iota(jnp.int32, (_N, _TS), 1)
    oh_f = (rows == lane_ts).astype(jnp.float32)
    mx = jnp.max(rlog, axis=1, keepdims=True)
    lse = mx[:, 0] + jnp.log(jnp.sum(jnp.exp(rlog - mx), axis=1))
    tgt = jnp.sum(oh_f * rlog, axis=1)
    lrow_ref[:] = jnp.reshape(jnp.sum(lse - tgt) * (1.0 / _N), (1, 1))

    offs = jnp.sum((rows < lane_ts).astype(jnp.float32), axis=0,
                   keepdims=True)
    gpos_col = jax.lax.broadcasted_iota(jnp.int32, (_N, 1), 0)
    gpos_f = gpos_col.astype(jnp.float32)
    rs = jnp.sum(((gpos_f - offs) >= 0.0).astype(jnp.float32),
                 axis=1, keepdims=True) - 1.0
    rs_prev = jnp.sum(((gpos_f - 1.0 - offs) >= 0.0).astype(jnp.float32),
                      axis=1, keepdims=True) - 1.0
    qs = jnp.floor(rs * (1.0 / _K))
    qs_prev = jnp.floor(rs_prev * (1.0 / _K))
    isnew_f = jnp.maximum((gpos_col % _TM == 0).astype(jnp.float32),
                          (qs != qs_prev).astype(jnp.float32))

    lane_u = jax.lax.broadcasted_iota(jnp.int32, (_N, _TS + 8), 1)
    oh_ext_f = (rows == lane_u).astype(jnp.float32)
    u = (oh_ext_f
         + (lane_u == _TS).astype(jnp.float32) * isnew_f).astype(jnp.bfloat16)
    io_i = jax.lax.broadcasted_iota(jnp.int32, (_CB, _CB), 0)
    io_j = jax.lax.broadcasted_iota(jnp.int32, (_CB, _CB), 1)
    Lt = (io_i >= io_j).astype(jnp.bfloat16)
    blocks = []
    carry = jnp.zeros((1, _TS + 8), jnp.float32)
    for b in range(_NB):
        ub = u[b * _CB:(b + 1) * _CB, :]
        cb = jax.lax.dot_general(Lt, ub, (((1,), (0,)), ((), ())),
                                 preferred_element_type=jnp.float32) + carry
        blocks.append(cb)
        carry = cb[_CB - 1:_CB, :]
    CU = jnp.concatenate(blocks, axis=0)

    rank = jnp.sum(oh_f * CU[:, :_TS], axis=1, keepdims=True) - 1.0
    segid = CU[:, _TS:_TS + 1] - 1.0
    offs_own = jnp.sum(oh_f * offs, axis=1, keepdims=True)
    pos = offs_own + rank

    io_pos = jax.lax.broadcasted_iota(jnp.int32, (_N, _N), 1)
    PT = (pos == io_pos.astype(jnp.float32)).astype(jnp.bfloat16)
    hss_ref[:] = jax.lax.dot_general(
        PT, hs_b, (((0,), (0,)), ((), ())),
        preferred_element_type=jnp.float32).astype(jnp.bfloat16)
    lane_m = jax.lax.broadcasted_iota(jnp.int32, (_N, _TS), 1)
    Xs = ((lane_m == 0).astype(jnp.float32)
          * cols.astype(jnp.float32)).astype(jnp.bfloat16)
    colss = jax.lax.dot_general(PT, Xs, (((0,), (0,)), ((), ())),
                                preferred_element_type=jnp.float32)
    cols_ref[:] = colss[:, 0:1].astype(jnp.int32)
    rows_ref[:] = rs.astype(jnp.int32)

    lane_sg = jax.lax.broadcasted_iota(jnp.int32, (_N, _NSEG), 1)
    Qc = (segid == lane_sg.astype(jnp.float32)).astype(jnp.bfloat16)
    Qt = Qc * isnew_f.astype(jnp.bfloat16)
    ihi = (gpos_col // 16).astype(jnp.float32)
    ilo = (gpos_col % 16).astype(jnp.float32)
    l0 = (lane_m == 0).astype(jnp.float32)
    l1 = (lane_m == 1).astype(jnp.float32)
    l2 = (lane_m == 2).astype(jnp.float32)
    l3 = (lane_m == 3).astype(jnp.float32)
    M1 = (l0 * ihi + l1 * ilo + l2 * qs + l3).astype(jnp.bfloat16)
    o1 = jax.lax.dot_general(Qt, M1, (((0,), (0,)), ((), ())),
                             preferred_element_type=jnp.float32)
    o2 = jax.lax.dot_general(Qc, M1, (((0,), (0,)), ((), ())),
                             preferred_element_type=jnp.float32)
    start = 16.0 * o1[:, 0:1] + o1[:, 1:2]
    quad = o1[:, 2:3]
    count = o2[:, 3:4]
    tile = jnp.floor(start * (1.0 / _TM))
    pad_f = (count == 0.0).astype(jnp.float32)
    tile = pad_f * float(_NT - 1) + (1.0 - pad_f) * tile
    qfetch = (1.0 - pad_f) * quad
    qcmp = pad_f * float(_SENT) + (1.0 - pad_f) * quad
    lane_meta = jax.lax.broadcasted_iota(jnp.int32, (_NSEG, _TS), 1)
    s0 = (lane_meta == 0).astype(jnp.float32)
    s1 = (lane_meta == 1).astype(jnp.float32)
    s2 = (lane_meta == 2).astype(jnp.float32)
    meta = s0 * qfetch + s1 * qcmp + s2 * tile
    meta_ref[:] = meta.astype(jnp.int32)


def _group_kernel(qf_ref, qc_ref, st_ref,
                  hss_ref, cw_ref, cb_ref, rows_ref, colss_ref,
                  out_ref, acc_ref):
    s = pl.program_id(0)
    q = qc_ref[s]
    tile = st_ref[s]
    rows_t = rows_ref[:]
    hs_t = hss_ref[:]
    cb = cb_ref[0]
    val = jnp.zeros((_TM, _TS), jnp.float32)
    for k in range(_K):
        ck = jnp.dot(hs_t, cw_ref[k].astype(jnp.bfloat16),
                     preferred_element_type=jnp.float32) + cb[k:k + 1, :]
        val += jnp.where(rows_t == q * _K + k, ck, 0.0)

    prev_tile = st_ref[jnp.maximum(s - 1, 0)]
    first = jnp.logical_or(s == 0, tile != prev_tile)
    next_tile = st_ref[jnp.minimum(s + 1, _NSEG - 1)]
    last = jnp.logical_or(s == _NSEG - 1, tile != next_tile)

    @pl.when(first)
    def _init():
        acc_ref[:] = val

    @pl.when(jnp.logical_not(first))
    def _accum():
        acc_ref[:] += val

    @pl.when(s == 0)
    def _zero_out():
        out_ref[:] = jnp.zeros_like(out_ref)

    @pl.when(last)
    def _tile_ce():
        clog = acc_ref[:]
        iot = jax.lax.broadcasted_iota(jnp.int32, (_TM, _TS), 1)
        onehot_f = (colss_ref[:] == iot).astype(jnp.float32)
        mx = jnp.max(clog, axis=1, keepdims=True)
        lse = mx[:, 0] + jnp.log(jnp.sum(jnp.exp(clog - mx), axis=1))
        tgt = jnp.sum(onehot_f * clog, axis=1)
        out_ref[:] += jnp.reshape(jnp.sum(lse - tgt) * (1.0 / _N), (1, 1))


def kernel(hidden_states, target_ids, W_row, b_row, col_weight, col_bias):
    hs = hidden_states.reshape(_N, _D)
    tid = target_ids.reshape(_N, 1)
    brow = b_row.reshape(1, _TS)

    hss, rws, colss, lrow, meta = pl.pallas_call(
        _route_kernel,
        in_specs=[
            pl.BlockSpec((_N, 1), lambda: (0, 0)),
            pl.BlockSpec((_N, _D), lambda: (0, 0)),
            pl.BlockSpec((_TS, _D), lambda: (0, 0)),
            pl.BlockSpec((1, _TS), lambda: (0, 0)),
        ],
        out_specs=[
            pl.BlockSpec((_N, _D), lambda: (0, 0)),
            pl.BlockSpec((_N, 1), lambda: (0, 0)),
            pl.BlockSpec((_N, 1), lambda: (0, 0)),
            pl.BlockSpec((1, 1), lambda: (0, 0)),
            pl.BlockSpec((_NSEG, _TS), lambda: (0, 0)),
        ],
        out_shape=[
            jax.ShapeDtypeStruct((_N, _D), jnp.bfloat16),
            jax.ShapeDtypeStruct((_N, 1), jnp.int32),
            jax.ShapeDtypeStruct((_N, 1), jnp.int32),
            jax.ShapeDtypeStruct((1, 1), jnp.float32),
            jax.ShapeDtypeStruct((_NSEG, _TS), jnp.int32),
        ],
    )(tid, hs, W_row, brow)

    qf = meta[:, 0]
    qc = meta[:, 1]
    st = meta[:, 2]

    lcol = pl.pallas_call(
        _group_kernel,
        grid_spec=pltpu.PrefetchScalarGridSpec(
            num_scalar_prefetch=3,
            grid=(_NSEG,),
            in_specs=[
                pl.BlockSpec((_TM, _D), lambda s, qf, qc, st: (st[s], 0)),
                pl.BlockSpec((_K, _D, _TS),
                             lambda s, qf, qc, st: (qf[s], 0, 0)),
                pl.BlockSpec((1, _K, _TS),
                             lambda s, qf, qc, st: (qf[s], 0, 0)),
                pl.BlockSpec((_TM, 1), lambda s, qf, qc, st: (st[s], 0)),
                pl.BlockSpec((_TM, 1), lambda s, qf, qc, st: (st[s], 0)),
            ],
            out_specs=pl.BlockSpec((1, 1), lambda s, qf, qc, st: (0, 0)),
            scratch_shapes=[pltpu.VMEM((_TM, _TS), jnp.float32)],
        ),
        out_shape=jax.ShapeDtypeStruct((1, 1), jnp.float32),
    )(qf, qc, st, hss, col_weight, col_bias.reshape(_NQ, _K, _TS),
      rws, colss)

    return (lrow + lcol)[0, 0]
